# baseline (speedup 1.0000x reference)
import functools

import jax
import jax.numpy as jnp
from jax import lax
from jax.experimental import pallas as pl
from jax.experimental.pallas import tpu as pltpu
from jax.experimental.pallas import tpu_sc as plsc

L = 16
NC = 2
NS = 16
CHUNK = 80


def _linear_kernel(x_ref, wt_ref, b_ref, o_ref):
    o_ref[...] = (
        jnp.dot(x_ref[...], wt_ref[...], preferred_element_type=jnp.float32)
        + b_ref[...]
    ).astype(jnp.bfloat16)


def _combine_kernel(p0u_ref, p1u_ref, p0i_ref, p1i_ref, ou_ref, oi_ref):
    ou_ref[...] = p0u_ref[...] + p1u_ref[...]
    oi_ref[...] = p0i_ref[...] + p1i_ref[...]


def _make_scatter(n_nodes, d, n_edges):
    n_workers = NC * NS
    e_per_w = n_edges // n_workers
    n_chunks = e_per_w // CHUNK
    zrows = CHUNK
    stride = ((n_nodes + NS - 1) // NS + zrows - 1) // zrows * zrows
    last_rows = n_nodes - (NS - 1) * zrows * (stride // zrows)
    last_rows = n_nodes - (NS - 1) * stride
    assert e_per_w * n_workers == n_edges
    assert n_chunks * CHUNK == e_per_w and n_chunks >= 6
    assert 0 < last_rows <= stride
    assert stride % zrows == 0 and last_rows % zrows == 0

    mesh = plsc.VectorSubcoreMesh(core_axis_name="c", subcore_axis_name="s",
                                  num_cores=NC, num_subcores=NS)

    @functools.partial(
        pl.kernel,
        out_type=jax.ShapeDtypeStruct((2 * n_nodes, d), jnp.float32),
        mesh=mesh,
        compiler_params=pltpu.CompilerParams(needs_layout_passes=False,
                                             use_tc_tiling_on_sc=False),
        scratch_types=(
            [pltpu.VMEM((CHUNK,), jnp.int32) for _ in range(4)]
            + [pltpu.VMEM((CHUNK,), jnp.int32) for _ in range(4)]
            + [pltpu.VMEM((CHUNK,), jnp.float32) for _ in range(4)]
            + [pltpu.VMEM((CHUNK, d // 2), jnp.int32) for _ in range(4)]
            + [pltpu.VMEM((CHUNK, d), jnp.float32) for _ in range(2)]
            + [pltpu.VMEM_SHARED((NS * stride, d), jnp.float32)]
            + [pltpu.SemaphoreType.DMA for _ in range(12)]
        ),
    )
    def scatter(embs_hbm, src_hbm, dst_hbm, w_hbm, feat_hbm, out_hbm,
                *refs):
        srcvs = refs[0:4]
        dstvs = refs[4:8]
        wrs = refs[8:12]
        gbufs = refs[12:16]
        sbufs = refs[16:18]
        acc = refs[18]
        isems = refs[19:23]
        gsems = refs[23:27]
        ssems = refs[27:31]

        c = lax.axis_index("c")
        s = lax.axis_index("s")

        r0 = s * stride

        @pl.when(c == 0)
        def _():
            @pl.when(s < NS - 1)
            def _():
                pltpu.sync_copy(feat_hbm.at[pl.ds(r0, stride)],
                                acc.at[pl.ds(r0, stride)])

            @pl.when(s == NS - 1)
            def _():
                pltpu.sync_copy(feat_hbm.at[pl.ds(r0, last_rows)],
                                acc.at[pl.ds(r0, last_rows)])

        @pl.when(c != 0)
        def _():
            zero = jnp.zeros((L,), jnp.float32)
            zb = sbufs[0]

            def zrow(i, carry):
                for j in range(d // L):
                    zb[i, pl.ds(j * L, L)] = zero
                return carry

            lax.fori_loop(0, zrows, zrow, 0)

            @pl.when(s < NS - 1)
            def _():
                for t in range(stride // zrows):
                    pltpu.sync_copy(zb, acc.at[pl.ds(r0 + t * zrows, zrows)])

            @pl.when(s == NS - 1)
            def _():
                for t in range(last_rows // zrows):
                    pltpu.sync_copy(zb, acc.at[pl.ds(r0 + t * zrows, zrows)])

        plsc.subcore_barrier()

        base = (c * NS + s) * e_per_w

        def wait_gat(dst_ref, sem):
            pltpu.make_async_copy(embs_hbm.at[pl.ds(0, CHUNK)],
                                  dst_ref, sem).wait()

        def wait_sct(dst_ref, sem):
            pltpu.make_async_copy(feat_hbm.at[pl.ds(0, CHUNK)],
                                  dst_ref, sem).wait()

        def wait_idx(dst_ref, sem):
            pltpu.make_async_copy(src_hbm.at[pl.ds(0, CHUNK)],
                                  dst_ref, sem).wait()

        gdn = lax.GatherDimensionNumbers(
            offset_dims=(), collapsed_slice_dims=(0,), start_index_map=(0,))

        def do_scale(gb, sb, wv):
            def kgroup(k, kcarry):
                row0 = k * L
                w16 = wv[pl.ds(row0, L)]
                for rr in range(L):
                    wb = lax.gather(
                        w16, jnp.full((L, 1), rr, jnp.int32), gdn,
                        slice_sizes=(1,),
                        mode=lax.GatherScatterMode.PROMISE_IN_BOUNDS)
                    for j in range(d // (2 * L)):
                        vi = gb[row0 + rr, pl.ds(j * L, L)]
                        a = lax.bitcast_convert_type(vi << 16, jnp.float32)
                        b2 = lax.bitcast_convert_type(
                            vi & jnp.int32(-65536), jnp.float32)
                        sb[row0 + rr, pl.ds(j * 2 * L, L)] = a * wb
                        sb[row0 + rr, pl.ds(j * 2 * L + L, L)] = b2 * wb
                return kcarry

            lax.fori_loop(0, CHUNK // L, kgroup, 0)

        for x in (0, 1):
            e = base + x * CHUNK
            pltpu.async_copy(src_hbm.at[pl.ds(e, CHUNK)], srcvs[x], isems[x])
            pltpu.async_copy(dst_hbm.at[pl.ds(e, CHUNK)], dstvs[x], isems[x])
            pltpu.async_copy(w_hbm.at[pl.ds(e, CHUNK)], wrs[x], isems[x])
        for x in (2, 3):
            e = base + x * CHUNK
            pltpu.async_copy(src_hbm.at[pl.ds(e, CHUNK)], srcvs[x], isems[x])
            pltpu.async_copy(w_hbm.at[pl.ds(e, CHUNK)], wrs[x], isems[x])
        for x in (0, 1):
            for _ in range(3):
                wait_idx(srcvs[x], isems[x])
            pltpu.async_copy(embs_hbm.at[srcvs[x]], gbufs[x], gsems[x])

        def body(g, carry):
            u = lax.rem(g, 4)

            for si in range(4):
                s2 = (si + 2) % 4

                @pl.when((g >= 2) & (u == si))
                def _(sb=sbufs[si % 2], sem=ssems[si % 2]):
                    wait_sct(sb, sem)

                @pl.when((g + 2 < n_chunks) & (u == si))
                def _(sv=srcvs[s2], gb=gbufs[s2], isem=isems[s2],
                      gsem=gsems[s2]):
                    wait_idx(sv, isem)
                    wait_idx(sv, isem)
                    pltpu.async_copy(embs_hbm.at[sv], gb, gsem)

            for si in range(4):
                @pl.when(u == si)
                def _(gb=gbufs[si], sb=sbufs[si % 2], gsem=gsems[si],
                      wv=wrs[si]):
                    wait_gat(gb, gsem)
                    do_scale(gb, sb, wv)

            for si in range(4):
                @pl.when(u == si)
                def _(sv=srcvs[si], dv=dstvs[si], sb=sbufs[si % 2],
                      isem=isems[si], ssem=ssems[si % 2]):
                    @pl.when(g >= 2)
                    def _():
                        wait_idx(sv, isem)
                    pltpu.async_copy(sb, acc.at[dv], ssem, add=True)

            e4 = base + (g + 4) * CHUNK
            e2 = base + (g + 2) * CHUNK
            for si in range(4):
                s2 = (si + 2) % 4

                @pl.when((g + 4 < n_chunks) & (u == si))
                def _(sv=srcvs[si], wv=wrs[si], isem=isems[si]):
                    pltpu.async_copy(src_hbm.at[pl.ds(e4, CHUNK)], sv, isem)
                    pltpu.async_copy(w_hbm.at[pl.ds(e4, CHUNK)], wv, isem)

                @pl.when((g + 2 < n_chunks) & (u == si))
                def _(dv=dstvs[s2], isem=isems[s2]):
                    pltpu.async_copy(dst_hbm.at[pl.ds(e2, CHUNK)], dv, isem)

            return carry

        lax.fori_loop(0, n_chunks, body, 0)

        for x in (n_chunks - 2, n_chunks - 1):
            wait_sct(sbufs[x % 2], ssems[x % 2])

        plsc.subcore_barrier()

        @pl.when(s < NS - 1)
        def _():
            pltpu.sync_copy(acc.at[pl.ds(r0, stride)],
                            out_hbm.at[pl.ds(c * n_nodes + r0, stride)])

        @pl.when(s == NS - 1)
        def _():
            pltpu.sync_copy(acc.at[pl.ds(r0, last_rows)],
                            out_hbm.at[pl.ds(c * n_nodes + r0, last_rows)])

    return scatter


def kernel(edge_index, edge_weight, user_feat, item_feat, W, b):
    n_users, d = user_feat.shape
    n_items = item_feat.shape[0]
    n_nodes = n_users + n_items
    n_edges = edge_weight.shape[0]

    feat_all = jnp.concatenate([user_feat, item_feat], axis=0)
    dst = edge_index[0]
    src = edge_index[1]

    perm = []
    for grp in range(d // (2 * L)):
        for t in range(L):
            perm.extend((grp * 2 * L + t, grp * 2 * L + L + t))
    perm = jnp.array(perm, jnp.int32)
    wt_perm = W.T[:, perm]
    b_perm = b[perm]

    blk = 2000
    embs = pl.pallas_call(
        _linear_kernel,
        grid=(n_nodes // blk,),
        in_specs=[
            pl.BlockSpec((blk, d), lambda i: (i, 0)),
            pl.BlockSpec((d, d), lambda i: (0, 0)),
            pl.BlockSpec((1, d), lambda i: (0, 0)),
        ],
        out_specs=pl.BlockSpec((blk, d), lambda i: (i, 0)),
        out_shape=jax.ShapeDtypeStruct((n_nodes, d), jnp.bfloat16),
    )(feat_all, wt_perm, b_perm.reshape(1, d))

    embs32 = lax.bitcast_convert_type(
        embs.reshape(n_nodes, d // 2, 2), jnp.int32)
    partials = _make_scatter(n_nodes, d, n_edges)(
        embs32, src, dst, edge_weight, feat_all)

    cblk = 1000
    gu = n_users // cblk
    gn = n_nodes // cblk
    out_user, out_item = pl.pallas_call(
        _combine_kernel,
        grid=(gu,),
        in_specs=[
            pl.BlockSpec((cblk, d), lambda i: (i, 0)),
            pl.BlockSpec((cblk, d), lambda i: (i + gn, 0)),
            pl.BlockSpec((cblk, d), lambda i: (i + gu, 0)),
            pl.BlockSpec((cblk, d), lambda i: (i + gn + gu, 0)),
        ],
        out_specs=[
            pl.BlockSpec((cblk, d), lambda i: (i, 0)),
            pl.BlockSpec((cblk, d), lambda i: (i, 0)),
        ],
        out_shape=[
            jax.ShapeDtypeStruct((n_users, d), jnp.float32),
            jax.ShapeDtypeStruct((n_items, d), jnp.float32),
        ],
    )(partials, partials, partials, partials)

    return (out_user, out_item)


# speedup vs baseline: 2.3043x; 2.3043x over previous
import functools

import jax
import jax.numpy as jnp
from jax import lax
from jax.experimental import pallas as pl
from jax.experimental.pallas import tpu as pltpu
from jax.experimental.pallas import tpu_sc as plsc

L = 16
NC = 2
NS = 16
CHUNK = 80


def _linear_kernel(x_ref, w_ref, b_ref, o_ref):
    o_ref[...] = (
        lax.dot_general(x_ref[...], w_ref[...], (((1,), (1,)), ((), ())),
                        preferred_element_type=jnp.float32)
        + b_ref[...]
    )


def _linear2_kernel(x_ref, w_ref, b_ref, prev_ref, o_ref):
    del prev_ref
    o_ref[...] = (
        lax.dot_general(x_ref[...], w_ref[...], (((1,), (1,)), ((), ())),
                        preferred_element_type=jnp.float32)
        + b_ref[...]
    )


def _combine_kernel(p0u_ref, p1u_ref, p0i_ref, p1i_ref, uf_ref, if_ref,
                    ou_ref, oi_ref):
    ou_ref[...] = p0u_ref[...] + p1u_ref[...] + uf_ref[...]
    oi_ref[...] = p0i_ref[...] + p1i_ref[...] + if_ref[...]


def _make_scatter(n_nodes, d, n_edges):
    n_workers = NC * NS
    e_per_w = n_edges // n_workers
    n_chunks = e_per_w // CHUNK
    zrows = CHUNK
    stride = ((n_nodes + NS - 1) // NS + zrows - 1) // zrows * zrows
    last_rows = n_nodes - (NS - 1) * zrows * (stride // zrows)
    last_rows = n_nodes - (NS - 1) * stride
    assert e_per_w * n_workers == n_edges
    assert n_chunks * CHUNK == e_per_w and n_chunks >= 6
    assert 0 < last_rows <= stride
    assert stride % zrows == 0 and last_rows % zrows == 0

    mesh = plsc.VectorSubcoreMesh(core_axis_name="c", subcore_axis_name="s",
                                  num_cores=NC, num_subcores=NS)

    @functools.partial(
        pl.kernel,
        out_type=jax.ShapeDtypeStruct((2 * n_nodes, d), jnp.float32),
        mesh=mesh,
        scratch_types=(
            [pltpu.VMEM((CHUNK,), jnp.int32) for _ in range(4)]
            + [pltpu.VMEM((CHUNK,), jnp.int32) for _ in range(4)]
            + [pltpu.VMEM((CHUNK,), jnp.float32) for _ in range(4)]
            + [pltpu.VMEM((CHUNK, d), jnp.float32) for _ in range(4)]
            + [pltpu.VMEM_SHARED((NS * stride, d), jnp.float32)]
            + [pltpu.SemaphoreType.DMA for _ in range(12)]
        ),
    )
    def scatter(embs_hbm, ei_hbm, w_hbm, out_hbm, *refs):
        srcvs = refs[0:4]
        dstvs = refs[4:8]
        wrs = refs[8:12]
        bufs = refs[12:16]
        acc = refs[16]
        isems = refs[17:21]
        gsems = refs[21:25]
        ssems = refs[25:29]

        c = lax.axis_index("c")
        s = lax.axis_index("s")

        r0 = s * stride

        zero = jnp.zeros((L,), jnp.float32)
        zb = bufs[0]

        def zrow(i, carry):
            for j in range(d // L):
                zb[i, pl.ds(j * L, L)] = zero
            return carry

        lax.fori_loop(0, zrows, zrow, 0)

        @pl.when(s < NS - 1)
        def _():
            for t in range(stride // zrows):
                pltpu.sync_copy(zb, acc.at[pl.ds(r0 + t * zrows, zrows)])

        @pl.when(s == NS - 1)
        def _():
            for t in range(last_rows // zrows):
                pltpu.sync_copy(zb, acc.at[pl.ds(r0 + t * zrows, zrows)])

        plsc.subcore_barrier()

        base = (c * NS + s) * e_per_w

        def wait_rows(dst_ref, sem):
            pltpu.make_async_copy(embs_hbm.at[pl.ds(0, CHUNK)],
                                  dst_ref, sem).wait()

        def wait_idx(dst_ref, sem):
            pltpu.make_async_copy(ei_hbm.at[pl.ds(0, CHUNK)],
                                  dst_ref, sem).wait()

        gdn = lax.GatherDimensionNumbers(
            offset_dims=(), collapsed_slice_dims=(0,), start_index_map=(0,))

        def do_scale(bf, wv):
            def kgroup(k, kcarry):
                row0 = k * L
                w16 = wv[pl.ds(row0, L)]
                for rr in range(L):
                    wb = lax.gather(
                        w16, jnp.full((L, 1), rr, jnp.int32), gdn,
                        slice_sizes=(1,),
                        mode=lax.GatherScatterMode.PROMISE_IN_BOUNDS)
                    for j in range(d // L):
                        sl = pl.ds(j * L, L)
                        bf[row0 + rr, sl] = bf[row0 + rr, sl] * wb
                return kcarry

            lax.fori_loop(0, CHUNK // L, kgroup, 0)

        for x in (0, 1):
            e = base + x * CHUNK
            pltpu.async_copy(ei_hbm.at[pl.ds(n_edges + e, CHUNK)],
                             srcvs[x], isems[x])
            pltpu.async_copy(ei_hbm.at[pl.ds(e, CHUNK)], dstvs[x], isems[x])
            pltpu.async_copy(w_hbm.at[pl.ds(e, CHUNK)], wrs[x], isems[x])
        for x in (2, 3):
            e = base + x * CHUNK
            pltpu.async_copy(ei_hbm.at[pl.ds(n_edges + e, CHUNK)],
                             srcvs[x], isems[x])
            pltpu.async_copy(w_hbm.at[pl.ds(e, CHUNK)], wrs[x], isems[x])
        for x in (0, 1):
            for _ in range(3):
                wait_idx(srcvs[x], isems[x])
            pltpu.async_copy(embs_hbm.at[srcvs[x]], bufs[x], gsems[x])

        def body(g, carry):
            u = lax.rem(g, 4)

            for si in range(4):
                s2 = (si + 2) % 4

                @pl.when((g >= 2) & (u == si))
                def _(bf=bufs[s2], sem=ssems[s2]):
                    wait_rows(bf, sem)

                @pl.when((g + 2 < n_chunks) & (u == si))
                def _(sv=srcvs[s2], bf=bufs[s2], isem=isems[s2],
                      gsem=gsems[s2]):
                    wait_idx(sv, isem)
                    wait_idx(sv, isem)
                    pltpu.async_copy(embs_hbm.at[sv], bf, gsem)

            for si in range(4):
                @pl.when(u == si)
                def _(bf=bufs[si], gsem=gsems[si], wv=wrs[si]):
                    wait_rows(bf, gsem)
                    do_scale(bf, wv)

            for si in range(4):
                @pl.when(u == si)
                def _(sv=srcvs[si], dv=dstvs[si], bf=bufs[si],
                      isem=isems[si], ssem=ssems[si]):
                    @pl.when(g >= 2)
                    def _():
                        wait_idx(sv, isem)
                    pltpu.async_copy(bf, acc.at[dv], ssem, add=True)

            e4 = base + (g + 4) * CHUNK
            e2 = base + (g + 2) * CHUNK
            for si in range(4):
                s2 = (si + 2) % 4

                @pl.when((g + 4 < n_chunks) & (u == si))
                def _(sv=srcvs[si], wv=wrs[si], isem=isems[si]):
                    pltpu.async_copy(ei_hbm.at[pl.ds(n_edges + e4, CHUNK)],
                                     sv, isem)
                    pltpu.async_copy(w_hbm.at[pl.ds(e4, CHUNK)], wv, isem)

                @pl.when((g + 2 < n_chunks) & (u == si))
                def _(dv=dstvs[s2], isem=isems[s2]):
                    pltpu.async_copy(ei_hbm.at[pl.ds(e2, CHUNK)], dv, isem)

            return carry

        lax.fori_loop(0, n_chunks, body, 0)

        for x in (n_chunks - 2, n_chunks - 1):
            wait_rows(bufs[x % 4], ssems[x % 4])

        plsc.subcore_barrier()

        @pl.when(s < NS - 1)
        def _():
            pltpu.sync_copy(acc.at[pl.ds(r0, stride)],
                            out_hbm.at[pl.ds(c * n_nodes + r0, stride)])

        @pl.when(s == NS - 1)
        def _():
            pltpu.sync_copy(acc.at[pl.ds(r0, last_rows)],
                            out_hbm.at[pl.ds(c * n_nodes + r0, last_rows)])

    return scatter


def kernel(edge_index, edge_weight, user_feat, item_feat, W, b):
    n_users, d = user_feat.shape
    n_items = item_feat.shape[0]
    n_nodes = n_users + n_items
    n_edges = edge_weight.shape[0]

    ei_flat = edge_index.reshape(2 * n_edges)

    blk = 1000
    gu = n_users // blk
    embs0 = pl.pallas_call(
        _linear_kernel,
        grid=(gu,),
        in_specs=[
            pl.BlockSpec((blk, d), lambda i: (i, 0)),
            pl.BlockSpec((d, d), lambda i: (0, 0)),
            pl.BlockSpec((1, d), lambda i: (0, 0)),
        ],
        out_specs=pl.BlockSpec((blk, d), lambda i: (i, 0)),
        out_shape=jax.ShapeDtypeStruct((n_nodes, d), jnp.float32),
    )(user_feat, W, b.reshape(1, d))
    embs = pl.pallas_call(
        _linear2_kernel,
        grid=(n_items // blk,),
        in_specs=[
            pl.BlockSpec((blk, d), lambda i: (i, 0)),
            pl.BlockSpec((d, d), lambda i: (0, 0)),
            pl.BlockSpec((1, d), lambda i: (0, 0)),
            pl.BlockSpec(memory_space=pl.ANY),
        ],
        out_specs=pl.BlockSpec((blk, d), lambda i: (i + gu, 0)),
        out_shape=jax.ShapeDtypeStruct((n_nodes, d), jnp.float32),
        input_output_aliases={3: 0},
    )(item_feat, W, b.reshape(1, d), embs0)

    partials = _make_scatter(n_nodes, d, n_edges)(embs, ei_flat, edge_weight)

    cblk = 1000
    gcu = n_users // cblk
    gn = n_nodes // cblk
    out_user, out_item = pl.pallas_call(
        _combine_kernel,
        grid=(gcu,),
        in_specs=[
            pl.BlockSpec((cblk, d), lambda i: (i, 0)),
            pl.BlockSpec((cblk, d), lambda i: (i + gn, 0)),
            pl.BlockSpec((cblk, d), lambda i: (i + gcu, 0)),
            pl.BlockSpec((cblk, d), lambda i: (i + gn + gcu, 0)),
            pl.BlockSpec((cblk, d), lambda i: (i, 0)),
            pl.BlockSpec((cblk, d), lambda i: (i, 0)),
        ],
        out_specs=[
            pl.BlockSpec((cblk, d), lambda i: (i, 0)),
            pl.BlockSpec((cblk, d), lambda i: (i, 0)),
        ],
        out_shape=[
            jax.ShapeDtypeStruct((n_users, d), jnp.float32),
            jax.ShapeDtypeStruct((n_items, d), jnp.float32),
        ],
    )(partials, partials, partials, partials, user_feat, item_feat)

    return (out_user, out_item)


# speedup vs baseline: 2.3054x; 1.0005x over previous
import functools

import jax
import jax.numpy as jnp
from jax import lax
from jax.experimental import pallas as pl
from jax.experimental.pallas import tpu as pltpu
from jax.experimental.pallas import tpu_sc as plsc

L = 16
NC = 2
NS = 16
CHUNK = 80


def _linear_kernel(x_ref, w_ref, b_ref, o_ref):
    o_ref[...] = (
        lax.dot_general(x_ref[...], w_ref[...], (((1,), (1,)), ((), ())),
                        preferred_element_type=jnp.float32)
        + b_ref[...]
    )


def _linear2_kernel(x_ref, w_ref, b_ref, prev_ref, o_ref):
    del prev_ref
    o_ref[...] = (
        lax.dot_general(x_ref[...], w_ref[...], (((1,), (1,)), ((), ())),
                        preferred_element_type=jnp.float32)
        + b_ref[...]
    )


def _combine_kernel(p0u_ref, p1u_ref, p0i_ref, p1i_ref, uf_ref, if_ref,
                    ou_ref, oi_ref):
    ou_ref[...] = p0u_ref[...] + p1u_ref[...] + uf_ref[...]
    oi_ref[...] = p0i_ref[...] + p1i_ref[...] + if_ref[...]


def _make_scatter(n_nodes, d, n_edges):
    n_workers = NC * NS
    e_per_w = n_edges // n_workers
    n_chunks = e_per_w // CHUNK
    zrows = CHUNK
    stride = ((n_nodes + NS - 1) // NS + zrows - 1) // zrows * zrows
    last_rows = n_nodes - (NS - 1) * zrows * (stride // zrows)
    last_rows = n_nodes - (NS - 1) * stride
    assert e_per_w * n_workers == n_edges
    assert n_chunks * CHUNK == e_per_w and n_chunks >= 6
    assert 0 < last_rows <= stride
    assert stride % zrows == 0 and last_rows % zrows == 0

    mesh = plsc.VectorSubcoreMesh(core_axis_name="c", subcore_axis_name="s",
                                  num_cores=NC, num_subcores=NS)

    @functools.partial(
        pl.kernel,
        out_type=jax.ShapeDtypeStruct((2 * n_nodes, d), jnp.float32),
        mesh=mesh,
        scratch_types=(
            [pltpu.VMEM((CHUNK,), jnp.int32) for _ in range(4)]
            + [pltpu.VMEM((CHUNK,), jnp.int32) for _ in range(4)]
            + [pltpu.VMEM((CHUNK,), jnp.float32) for _ in range(4)]
            + [pltpu.VMEM((CHUNK, d), jnp.float32) for _ in range(4)]
            + [pltpu.VMEM_SHARED((NS * stride, d), jnp.float32)]
            + [pltpu.SemaphoreType.DMA for _ in range(12)]
        ),
    )
    def scatter(embs_hbm, ei_hbm, w_hbm, out_hbm, *refs):
        srcvs = refs[0:4]
        dstvs = refs[4:8]
        wrs = refs[8:12]
        bufs = refs[12:16]
        acc = refs[16]
        isems = refs[17:21]
        gsems = refs[21:25]
        ssems = refs[25:29]

        c = lax.axis_index("c")
        s = lax.axis_index("s")

        r0 = s * stride

        zero = jnp.zeros((L,), jnp.float32)
        zb = bufs[0]

        def zrow(i, carry):
            for j in range(d // L):
                zb[i, pl.ds(j * L, L)] = zero
            return carry

        lax.fori_loop(0, zrows, zrow, 0)

        @pl.when(s < NS - 1)
        def _():
            for t in range(stride // zrows):
                pltpu.sync_copy(zb, acc.at[pl.ds(r0 + t * zrows, zrows)])

        @pl.when(s == NS - 1)
        def _():
            for t in range(last_rows // zrows):
                pltpu.sync_copy(zb, acc.at[pl.ds(r0 + t * zrows, zrows)])

        plsc.subcore_barrier()

        base = (c * NS + s) * e_per_w

        def wait_rows(dst_ref, sem):
            pltpu.make_async_copy(embs_hbm.at[pl.ds(0, CHUNK)],
                                  dst_ref, sem).wait()

        def wait_idx(dst_ref, sem):
            pltpu.make_async_copy(ei_hbm.at[pl.ds(0, CHUNK)],
                                  dst_ref, sem).wait()

        gdn = lax.GatherDimensionNumbers(
            offset_dims=(), collapsed_slice_dims=(0,), start_index_map=(0,))

        def do_scale(bf, wv):
            def kgroup(k, kcarry):
                row0 = k * L
                w16 = wv[pl.ds(row0, L)]
                for rr in range(L):
                    wb = lax.gather(
                        w16, jnp.full((L, 1), rr, jnp.int32), gdn,
                        slice_sizes=(1,),
                        mode=lax.GatherScatterMode.PROMISE_IN_BOUNDS)
                    for j in range(d // L):
                        sl = pl.ds(j * L, L)
                        bf[row0 + rr, sl] = bf[row0 + rr, sl] * wb
                return kcarry

            lax.fori_loop(0, CHUNK // L, kgroup, 0)

        for x in (0, 1):
            e = base + x * CHUNK
            pltpu.async_copy(ei_hbm.at[pl.ds(n_edges + e, CHUNK)],
                             srcvs[x], isems[x])
            pltpu.async_copy(ei_hbm.at[pl.ds(e, CHUNK)], dstvs[x], isems[x])
            pltpu.async_copy(w_hbm.at[pl.ds(e, CHUNK)], wrs[x], isems[x])
        for x in (2, 3):
            e = base + x * CHUNK
            pltpu.async_copy(ei_hbm.at[pl.ds(n_edges + e, CHUNK)],
                             srcvs[x], isems[x])
            pltpu.async_copy(w_hbm.at[pl.ds(e, CHUNK)], wrs[x], isems[x])
        for x in (0, 1):
            for _ in range(3):
                wait_idx(srcvs[x], isems[x])
            pltpu.async_copy(embs_hbm.at[srcvs[x]], bufs[x], gsems[x])

        def body(g, carry):
            u = lax.rem(g, 4)
            e4 = base + (g + 4) * CHUNK
            e2 = base + (g + 2) * CHUNK

            for si in range(4):
                s2 = (si + 2) % 4

                @pl.when(u == si)
                def _(sv=srcvs[si], dv=dstvs[si], bf=bufs[si],
                      isem=isems[si], gsem=gsems[si], ssem=ssems[si],
                      wv=wrs[si], sv2=srcvs[s2], dv2=dstvs[s2], bf2=bufs[s2],
                      isem2=isems[s2], gsem2=gsems[s2], ssem2=ssems[s2]):
                    @pl.when(g >= 2)
                    def _():
                        wait_rows(bf2, ssem2)

                    @pl.when(g + 2 < n_chunks)
                    def _():
                        wait_idx(sv2, isem2)
                        wait_idx(sv2, isem2)
                        pltpu.async_copy(embs_hbm.at[sv2], bf2, gsem2)

                    wait_rows(bf, gsem)
                    do_scale(bf, wv)

                    @pl.when(g >= 2)
                    def _():
                        wait_idx(sv, isem)

                    pltpu.async_copy(bf, acc.at[dv], ssem, add=True)

                    @pl.when(g + 4 < n_chunks)
                    def _():
                        pltpu.async_copy(
                            ei_hbm.at[pl.ds(n_edges + e4, CHUNK)], sv, isem)
                        pltpu.async_copy(w_hbm.at[pl.ds(e4, CHUNK)], wv, isem)

                    @pl.when(g + 2 < n_chunks)
                    def _():
                        pltpu.async_copy(ei_hbm.at[pl.ds(e2, CHUNK)],
                                         dv2, isem2)

            return carry

        lax.fori_loop(0, n_chunks, body, 0)

        for x in (n_chunks - 2, n_chunks - 1):
            wait_rows(bufs[x % 4], ssems[x % 4])

        plsc.subcore_barrier()

        @pl.when(s < NS - 1)
        def _():
            pltpu.sync_copy(acc.at[pl.ds(r0, stride)],
                            out_hbm.at[pl.ds(c * n_nodes + r0, stride)])

        @pl.when(s == NS - 1)
        def _():
            pltpu.sync_copy(acc.at[pl.ds(r0, last_rows)],
                            out_hbm.at[pl.ds(c * n_nodes + r0, last_rows)])

    return scatter


def kernel(edge_index, edge_weight, user_feat, item_feat, W, b):
    n_users, d = user_feat.shape
    n_items = item_feat.shape[0]
    n_nodes = n_users + n_items
    n_edges = edge_weight.shape[0]

    ei_flat = edge_index.reshape(2 * n_edges)

    blk = 1000
    gu = n_users // blk
    embs0 = pl.pallas_call(
        _linear_kernel,
        grid=(gu,),
        in_specs=[
            pl.BlockSpec((blk, d), lambda i: (i, 0)),
            pl.BlockSpec((d, d), lambda i: (0, 0)),
            pl.BlockSpec((1, d), lambda i: (0, 0)),
        ],
        out_specs=pl.BlockSpec((blk, d), lambda i: (i, 0)),
        out_shape=jax.ShapeDtypeStruct((n_nodes, d), jnp.float32),
    )(user_feat, W, b.reshape(1, d))
    embs = pl.pallas_call(
        _linear2_kernel,
        grid=(n_items // blk,),
        in_specs=[
            pl.BlockSpec((blk, d), lambda i: (i, 0)),
            pl.BlockSpec((d, d), lambda i: (0, 0)),
            pl.BlockSpec((1, d), lambda i: (0, 0)),
            pl.BlockSpec(memory_space=pl.ANY),
        ],
        out_specs=pl.BlockSpec((blk, d), lambda i: (i + gu, 0)),
        out_shape=jax.ShapeDtypeStruct((n_nodes, d), jnp.float32),
        input_output_aliases={3: 0},
    )(item_feat, W, b.reshape(1, d), embs0)

    partials = _make_scatter(n_nodes, d, n_edges)(embs, ei_flat, edge_weight)

    cblk = 1000
    gcu = n_users // cblk
    gn = n_nodes // cblk
    out_user, out_item = pl.pallas_call(
        _combine_kernel,
        grid=(gcu,),
        in_specs=[
            pl.BlockSpec((cblk, d), lambda i: (i, 0)),
            pl.BlockSpec((cblk, d), lambda i: (i + gn, 0)),
            pl.BlockSpec((cblk, d), lambda i: (i + gcu, 0)),
            pl.BlockSpec((cblk, d), lambda i: (i + gn + gcu, 0)),
            pl.BlockSpec((cblk, d), lambda i: (i, 0)),
            pl.BlockSpec((cblk, d), lambda i: (i, 0)),
        ],
        out_specs=[
            pl.BlockSpec((cblk, d), lambda i: (i, 0)),
            pl.BlockSpec((cblk, d), lambda i: (i, 0)),
        ],
        out_shape=[
            jax.ShapeDtypeStruct((n_users, d), jnp.float32),
            jax.ShapeDtypeStruct((n_items, d), jnp.float32),
        ],
    )(partials, partials, partials, partials, user_feat, item_feat)

    return (out_user, out_item)


# speedup vs baseline: 2.3388x; 1.0145x over previous
import functools

import jax
import jax.numpy as jnp
from jax import lax
from jax.experimental import pallas as pl
from jax.experimental.pallas import tpu as pltpu
from jax.experimental.pallas import tpu_sc as plsc

L = 16
NC = 2
NS = 16
CHUNK = 80


def _linear_kernel(x_ref, w_ref, b_ref, o_ref):
    o_ref[...] = (
        lax.dot_general(x_ref[...], w_ref[...], (((1,), (1,)), ((), ())),
                        preferred_element_type=jnp.float32)
        + b_ref[...]
    )


def _linear2_kernel(x_ref, w_ref, b_ref, prev_ref, o_ref):
    del prev_ref
    o_ref[...] = (
        lax.dot_general(x_ref[...], w_ref[...], (((1,), (1,)), ((), ())),
                        preferred_element_type=jnp.float32)
        + b_ref[...]
    )


def _combine_kernel(p0u_ref, p1u_ref, p0i_ref, p1i_ref, uf_ref, if_ref,
                    ou_ref, oi_ref):
    ou_ref[...] = p0u_ref[...] + p1u_ref[...] + uf_ref[...]
    oi_ref[...] = p0i_ref[...] + p1i_ref[...] + if_ref[...]


def _make_scatter(n_nodes, d, n_edges):
    n_workers = NC * NS
    e_per_w = n_edges // n_workers
    n_chunks = e_per_w // CHUNK
    zrows = CHUNK
    stride = ((n_nodes + NS - 1) // NS + zrows - 1) // zrows * zrows
    last_rows = n_nodes - (NS - 1) * zrows * (stride // zrows)
    last_rows = n_nodes - (NS - 1) * stride
    assert e_per_w * n_workers == n_edges
    assert n_chunks * CHUNK == e_per_w and n_chunks >= 6
    assert 0 < last_rows <= stride
    assert stride % zrows == 0 and last_rows % zrows == 0

    mesh = plsc.VectorSubcoreMesh(core_axis_name="c", subcore_axis_name="s",
                                  num_cores=NC, num_subcores=NS)

    @functools.partial(
        pl.kernel,
        out_type=jax.ShapeDtypeStruct((2 * n_nodes, d), jnp.float32),
        mesh=mesh,
        scratch_types=(
            [pltpu.VMEM((CHUNK,), jnp.int32) for _ in range(4)]
            + [pltpu.VMEM((CHUNK,), jnp.int32) for _ in range(4)]
            + [pltpu.VMEM((CHUNK,), jnp.float32) for _ in range(4)]
            + [pltpu.VMEM((CHUNK, d), jnp.float32) for _ in range(4)]
            + [pltpu.VMEM_SHARED((NS * stride, d), jnp.float32)]
            + [pltpu.SemaphoreType.DMA for _ in range(12)]
        ),
    )
    def scatter(embs_hbm, ei_hbm, w_hbm, out_hbm, *refs):
        srcvs = refs[0:4]
        dstvs = refs[4:8]
        wrs = refs[8:12]
        bufs = refs[12:16]
        acc = refs[16]
        isems = refs[17:21]
        gsems = refs[21:25]
        ssems = refs[25:29]

        c = lax.axis_index("c")
        s = lax.axis_index("s")

        r0 = s * stride
        base = (c * NS + s) * e_per_w

        def wait_rows(dst_ref, sem):
            pltpu.make_async_copy(embs_hbm.at[pl.ds(0, CHUNK)],
                                  dst_ref, sem).wait()

        def wait_idx(dst_ref, sem):
            pltpu.make_async_copy(ei_hbm.at[pl.ds(0, CHUNK)],
                                  dst_ref, sem).wait()

        gdn = lax.GatherDimensionNumbers(
            offset_dims=(), collapsed_slice_dims=(0,), start_index_map=(0,))

        def do_scale(bf, wv):
            def kgroup(k, kcarry):
                row0 = k * L
                w16 = wv[pl.ds(row0, L)]
                for rr in range(L):
                    wb = lax.gather(
                        w16, jnp.full((L, 1), rr, jnp.int32), gdn,
                        slice_sizes=(1,),
                        mode=lax.GatherScatterMode.PROMISE_IN_BOUNDS)
                    for j in range(d // L):
                        sl = pl.ds(j * L, L)
                        bf[row0 + rr, sl] = bf[row0 + rr, sl] * wb
                return kcarry

            lax.fori_loop(0, CHUNK // L, kgroup, 0)

        for x in (0, 1):
            e = base + x * CHUNK
            pltpu.async_copy(ei_hbm.at[pl.ds(n_edges + e, CHUNK)],
                             srcvs[x], isems[x])
            pltpu.async_copy(ei_hbm.at[pl.ds(e, CHUNK)], dstvs[x], isems[x])
            pltpu.async_copy(w_hbm.at[pl.ds(e, CHUNK)], wrs[x], isems[x])
        for x in (2, 3):
            e = base + x * CHUNK
            pltpu.async_copy(ei_hbm.at[pl.ds(n_edges + e, CHUNK)],
                             srcvs[x], isems[x])
            pltpu.async_copy(w_hbm.at[pl.ds(e, CHUNK)], wrs[x], isems[x])
        for x in (0, 1):
            for _ in range(3):
                wait_idx(srcvs[x], isems[x])
            pltpu.async_copy(embs_hbm.at[srcvs[x]], bufs[x], gsems[x])

        zero = jnp.zeros((L,), jnp.float32)
        zb = bufs[3]

        def zrow(i, carry):
            for j in range(d // L):
                zb[i, pl.ds(j * L, L)] = zero
            return carry

        lax.fori_loop(0, zrows, zrow, 0)

        @pl.when(s < NS - 1)
        def _():
            for t in range(stride // zrows):
                pltpu.sync_copy(zb, acc.at[pl.ds(r0 + t * zrows, zrows)])

        @pl.when(s == NS - 1)
        def _():
            for t in range(last_rows // zrows):
                pltpu.sync_copy(zb, acc.at[pl.ds(r0 + t * zrows, zrows)])

        plsc.subcore_barrier()

        def body(g, carry):
            u = lax.rem(g, 4)
            e4 = base + (g + 4) * CHUNK
            e2 = base + (g + 2) * CHUNK

            for si in range(4):
                s2 = (si + 2) % 4

                @pl.when(u == si)
                def _(sv=srcvs[si], dv=dstvs[si], bf=bufs[si],
                      isem=isems[si], gsem=gsems[si], ssem=ssems[si],
                      wv=wrs[si], sv2=srcvs[s2], dv2=dstvs[s2], bf2=bufs[s2],
                      isem2=isems[s2], gsem2=gsems[s2], ssem2=ssems[s2]):
                    @pl.when(g >= 2)
                    def _():
                        wait_rows(bf2, ssem2)

                    @pl.when(g + 2 < n_chunks)
                    def _():
                        wait_idx(sv2, isem2)
                        wait_idx(sv2, isem2)
                        pltpu.async_copy(embs_hbm.at[sv2], bf2, gsem2)

                    wait_rows(bf, gsem)
                    do_scale(bf, wv)

                    @pl.when(g >= 2)
                    def _():
                        wait_idx(sv, isem)

                    pltpu.async_copy(bf, acc.at[dv], ssem, add=True)

                    @pl.when(g + 4 < n_chunks)
                    def _():
                        pltpu.async_copy(
                            ei_hbm.at[pl.ds(n_edges + e4, CHUNK)], sv, isem)
                        pltpu.async_copy(w_hbm.at[pl.ds(e4, CHUNK)], wv, isem)

                    @pl.when(g + 2 < n_chunks)
                    def _():
                        pltpu.async_copy(ei_hbm.at[pl.ds(e2, CHUNK)],
                                         dv2, isem2)

            return carry

        lax.fori_loop(0, n_chunks, body, 0)

        for x in (n_chunks - 2, n_chunks - 1):
            wait_rows(bufs[x % 4], ssems[x % 4])

        plsc.subcore_barrier()

        @pl.when(s < NS - 1)
        def _():
            pltpu.sync_copy(acc.at[pl.ds(r0, stride)],
                            out_hbm.at[pl.ds(c * n_nodes + r0, stride)])

        @pl.when(s == NS - 1)
        def _():
            pltpu.sync_copy(acc.at[pl.ds(r0, last_rows)],
                            out_hbm.at[pl.ds(c * n_nodes + r0, last_rows)])

    return scatter


def kernel(edge_index, edge_weight, user_feat, item_feat, W, b):
    n_users, d = user_feat.shape
    n_items = item_feat.shape[0]
    n_nodes = n_users + n_items
    n_edges = edge_weight.shape[0]

    ei_flat = edge_index.reshape(2 * n_edges)

    blk = 1000
    gu = n_users // blk
    embs0 = pl.pallas_call(
        _linear_kernel,
        grid=(gu,),
        in_specs=[
            pl.BlockSpec((blk, d), lambda i: (i, 0)),
            pl.BlockSpec((d, d), lambda i: (0, 0)),
            pl.BlockSpec((1, d), lambda i: (0, 0)),
        ],
        out_specs=pl.BlockSpec((blk, d), lambda i: (i, 0)),
        out_shape=jax.ShapeDtypeStruct((n_nodes, d), jnp.float32),
    )(user_feat, W, b.reshape(1, d))
    embs = pl.pallas_call(
        _linear2_kernel,
        grid=(n_items // blk,),
        in_specs=[
            pl.BlockSpec((blk, d), lambda i: (i, 0)),
            pl.BlockSpec((d, d), lambda i: (0, 0)),
            pl.BlockSpec((1, d), lambda i: (0, 0)),
            pl.BlockSpec(memory_space=pl.ANY),
        ],
        out_specs=pl.BlockSpec((blk, d), lambda i: (i + gu, 0)),
        out_shape=jax.ShapeDtypeStruct((n_nodes, d), jnp.float32),
        input_output_aliases={3: 0},
    )(item_feat, W, b.reshape(1, d), embs0)

    partials = _make_scatter(n_nodes, d, n_edges)(embs, ei_flat, edge_weight)

    cblk = 1000
    gcu = n_users // cblk
    gn = n_nodes // cblk
    out_user, out_item = pl.pallas_call(
        _combine_kernel,
        grid=(gcu,),
        in_specs=[
            pl.BlockSpec((cblk, d), lambda i: (i, 0)),
            pl.BlockSpec((cblk, d), lambda i: (i + gn, 0)),
            pl.BlockSpec((cblk, d), lambda i: (i + gcu, 0)),
            pl.BlockSpec((cblk, d), lambda i: (i + gn + gcu, 0)),
            pl.BlockSpec((cblk, d), lambda i: (i, 0)),
            pl.BlockSpec((cblk, d), lambda i: (i, 0)),
        ],
        out_specs=[
            pl.BlockSpec((cblk, d), lambda i: (i, 0)),
            pl.BlockSpec((cblk, d), lambda i: (i, 0)),
        ],
        out_shape=[
            jax.ShapeDtypeStruct((n_users, d), jnp.float32),
            jax.ShapeDtypeStruct((n_items, d), jnp.float32),
        ],
    )(partials, partials, partials, partials, user_feat, item_feat)

    return (out_user, out_item)
